# Initial kernel scaffold; baseline (speedup 1.0000x reference)
import jax
import jax.numpy as jnp
from jax.experimental import pallas as pl


def kernel(observations, pos_x, pos_y, feat_embed, feature_scale, W1, b1, ln_g, ln_b, W2, b2, W3, b3, Wa, ba, Wv, bv):
    raise NotImplementedError("write your pallas kernel here")


# speedup vs baseline: 25.0411x; 25.0411x over previous
import functools
import jax
import jax.numpy as jnp
from jax.experimental import pallas as pl
from jax.experimental.pallas import tpu as pltpu

H = 192
NBINS = 512
ROWS = 256


def _policy_kernel(coords_ref, feats_ref, vals_ref, fscale_ref, t2t_ref,
                   w1_ref, b1_ref, lng_ref, lnb_ref, w2_ref, b2_ref,
                   w3_ref, b3_ref, wav_ref, bav_ref,
                   out_ref, hist_ref, cnt_ref):
    T = coords_ref.shape[0]
    rows = coords_ref.shape[1]

    hist_ref[...] = jnp.zeros((NBINS, rows), jnp.float32)
    cnt_ref[...] = jnp.zeros((8, rows), jnp.float32)

    invs_row = (1.0 / (fscale_ref[...] + 1e-6)).reshape(1, 256)
    iota_col = jax.lax.broadcasted_iota(jnp.int32, (256, 1), 0)

    def body(t, _):
        c = coords_ref[pl.ds(t, 1), :]
        f = jnp.clip(feats_ref[pl.ds(t, 1), :], 0, 255)
        v = vals_ref[pl.ds(t, 1), :].astype(jnp.float32)
        valid = c != 255
        u = jnp.where(valid, v, 0.0)
        cbin = ((c >> 4) & 15) * 16 + (c & 15)

        eqf = (f == iota_col).astype(jnp.float32)
        isc = jnp.dot(invs_row, eqf,
                      preferred_element_type=jnp.float32)
        w = u * isc

        hist_ref[:256, :] += jnp.where(cbin == iota_col, w, 0.0)
        hist_ref[256:, :] += eqf * w
        cnt_ref[0:1, :] += jnp.where(valid, 1.0, 0.0)
        return 0

    jax.lax.fori_loop(0, T, body, 0)

    cnt = jnp.maximum(cnt_ref[0:1, :], 1.0)
    summary = jnp.dot(t2t_ref[...], hist_ref[...],
                      preferred_element_type=jnp.float32)
    summary = summary * jax.lax.rsqrt(cnt)

    h = jnp.maximum(jnp.dot(w1_ref[...], summary,
                            preferred_element_type=jnp.float32)
                    + b1_ref[...], 0.0)
    mu = jnp.mean(h, axis=0, keepdims=True)
    var = jnp.mean((h - mu) ** 2, axis=0, keepdims=True)
    h = (h - mu) * jax.lax.rsqrt(var + 1e-5) * lng_ref[...] + lnb_ref[...]
    h = jnp.maximum(jnp.dot(w2_ref[...], h,
                            preferred_element_type=jnp.float32)
                    + b2_ref[...], 0.0)
    h = jnp.maximum(jnp.dot(w3_ref[...], h,
                            preferred_element_type=jnp.float32)
                    + b3_ref[...], 0.0)
    out_ref[...] = jnp.dot(wav_ref[...], h,
                           preferred_element_type=jnp.float32) + bav_ref[...]


def kernel(observations, pos_x, pos_y, feat_embed, feature_scale, W1, b1,
           ln_g, ln_b, W2, b2, W3, b3, Wa, ba, Wv, bv):
    B, T, _ = observations.shape
    coords = observations[..., 0].T
    feats = observations[..., 1].T
    vals = observations[..., 2].T

    pos_xy = (pos_x[:16, None, :] + pos_y[None, :16, :]).reshape(256, H)
    t2t = jnp.concatenate([pos_xy, feat_embed], axis=0).T
    wavt = jnp.concatenate([Wa, Wv], axis=1).T
    bav = jnp.concatenate([ba, bv], axis=0).reshape(20, 1)
    nout = wavt.shape[0]

    rows = min(ROWS, B)
    grid = (B // rows,)
    tok_spec = pl.BlockSpec((T, rows), lambda i: (0, i))
    full = lambda shape: pl.BlockSpec(shape, lambda i: (0, 0))

    out = pl.pallas_call(
        _policy_kernel,
        grid=grid,
        in_specs=[
            tok_spec, tok_spec, tok_spec,
            full((1, 256)), full((H, NBINS)),
            full((H, H)), full((H, 1)), full((H, 1)), full((H, 1)),
            full((H, H)), full((H, 1)),
            full((H, H)), full((H, 1)),
            full((nout, H)), full((nout, 1)),
        ],
        out_specs=pl.BlockSpec((nout, rows), lambda i: (0, i)),
        out_shape=jax.ShapeDtypeStruct((nout, B), jnp.float32),
        scratch_shapes=[
            pltpu.VMEM((NBINS, rows), jnp.float32),
            pltpu.VMEM((8, rows), jnp.float32),
        ],
    )(coords, feats, vals, feature_scale.reshape(1, 256), t2t,
      W1.T, b1.reshape(H, 1), ln_g.reshape(H, 1), ln_b.reshape(H, 1),
      W2.T, b2.reshape(H, 1), W3.T, b3.reshape(H, 1), wavt, bav)

    outT = out.T
    l0 = outT[:, :9]
    l1 = outT[:, 9:19]
    values = outT[:, 19:20]
    return (l0, l1, values)
